# baseline (speedup 1.0000x reference)
import functools

import jax
import jax.numpy as jnp
from jax import lax
from jax.experimental import pallas as pl
from jax.experimental.pallas import tpu as pltpu
from jax.experimental.pallas import tpu_sc as plsc

DIM = 64
SCALE = 8.0

_NC = 2
_NS = 16
_NW = _NC * _NS

_C = 128
_NBUF = 4


@jax.jit
def _emb_call(xt3, lut):
    nj, ncb, _ = xt3.shape
    nunit = nj * ncb
    upw = nunit // _NW
    mesh = plsc.VectorSubcoreMesh(core_axis_name="c", subcore_axis_name="s")

    @functools.partial(
        pl.kernel,
        out_type=jax.ShapeDtypeStruct((nj, DIM // 8, ncb, 8, _C), jnp.float32),
        mesh=mesh,
        scratch_types=[
            pltpu.VMEM((_NBUF, _C), jnp.int32),
            pltpu.VMEM((_NBUF, _C, DIM), jnp.float32),
            pltpu.VMEM((_NBUF, DIM, _C), jnp.float32),
            pltpu.SemaphoreType.DMA((_NBUF,)),
            pltpu.SemaphoreType.DMA((_NBUF,)),
            pltpu.SemaphoreType.DMA((_NBUF,)),
        ],
        compiler_params=pltpu.CompilerParams(
            use_tc_tiling_on_sc=False, needs_layout_passes=False),
    )
    def emb(x_hbm, lut_hbm, out_hbm, idx_v, rows_v, tr_v, isem, gsem, osem):
        wid = lax.axis_index("s") * _NC + lax.axis_index("c")
        ubase = wid * upw
        iota = lax.iota(jnp.int32, 16)

        def unit_jc(u):
            ug = ubase + u
            return ug // ncb, ug % ncb

        def idx_copy(u, b):
            j, c = unit_jc(u)
            return pltpu.make_async_copy(x_hbm.at[j, c], idx_v.at[b], isem.at[b])

        def gather(b):
            return pltpu.make_async_copy(
                lut_hbm.at[idx_v.at[b]], rows_v.at[b], gsem.at[b])

        def out_copy(u, b, r):
            j, c = unit_jc(u)
            return pltpu.make_async_copy(
                tr_v.at[b, pl.ds(r * 8, 8)], out_hbm.at[j, r, c], osem.at[b])

        for k in range(_NBUF - 1):
            idx_copy(k, k).start()
        idx_copy(0, 0).wait()
        gather(0).start()

        @pl.loop(0, upw, step=_NBUF)
        def outer(u0):
            for b in range(_NBUF):
                u = u0 + b

                @pl.when(u + _NBUF - 1 < upw)
                def _():
                    idx_copy(u + _NBUF - 1, (b + _NBUF - 1) % _NBUF).start()

                @pl.when(u + 1 < upw)
                def _():
                    nb = (b + 1) % _NBUF
                    idx_copy(u + 1, nb).wait()
                    gather(nb).start()

                gather(b).wait()

                @pl.when(u >= _NBUF)
                def _():
                    for r in range(DIM // 8):
                        out_copy(u - _NBUF, b, r).wait()

                def dloop(d, carry):
                    col = jnp.full((16,), d, jnp.int32)
                    for g in range(_C // 16):
                        v = plsc.load_gather(
                            rows_v.at[b], [iota + (g * 16), col])
                        tr_v[b, d, pl.ds(g * 16, 16)] = v * SCALE
                    return carry

                lax.fori_loop(0, DIM, dloop, 0)

                for r in range(DIM // 8):
                    out_copy(u, b, r).start()

        for b in range(_NBUF):
            for r in range(DIM // 8):
                out_copy(upw - _NBUF + b, b, r).wait()

    return emb(xt3, lut)


def kernel(X, lut):
    s0, s1 = X.shape
    xt3 = jnp.transpose(X).astype(jnp.int32).reshape(s1, s0 // _C, _C)
    out5 = _emb_call(xt3, lut)
    out = out5.transpose(2, 4, 0, 1, 3).reshape(s0, s1, DIM)
    return out


# speedup vs baseline: 1.4764x; 1.4764x over previous
import functools

import jax
import jax.numpy as jnp
from jax import lax
from jax.experimental import pallas as pl
from jax.experimental.pallas import tpu as pltpu
from jax.experimental.pallas import tpu_sc as plsc

DIM = 64
SCALE = 8.0

_NC = 2
_NS = 16
_NW = _NC * _NS

_C = 128
_NBUF = 4


@jax.jit
def _emb_call(xt3, lut):
    nj, ncb, _ = xt3.shape
    nunit = nj * ncb
    upw = nunit // _NW
    mesh = plsc.VectorSubcoreMesh(core_axis_name="c", subcore_axis_name="s")

    @functools.partial(
        pl.kernel,
        out_type=jax.ShapeDtypeStruct((nj, DIM // 8, ncb, 8, _C), jnp.float32),
        mesh=mesh,
        scratch_types=[
            pltpu.VMEM((_NBUF, _C), jnp.int32),
            pltpu.VMEM((_NBUF, _C, DIM), jnp.float32),
            pltpu.VMEM((_NBUF, DIM, _C), jnp.float32),
            pltpu.SemaphoreType.DMA((_NBUF,)),
            pltpu.SemaphoreType.DMA((_NBUF,)),
            pltpu.SemaphoreType.DMA((_NBUF,)),
        ],
        compiler_params=pltpu.CompilerParams(
            use_tc_tiling_on_sc=False, needs_layout_passes=False),
    )
    def emb(x_hbm, lut_hbm, out_hbm, idx_v, rows_v, tr_v, isem, gsem, osem):
        wid = lax.axis_index("s") * _NC + lax.axis_index("c")
        ubase = wid * upw
        iota = lax.iota(jnp.int32, 16)

        def unit_jc(u):
            ug = ubase + u
            return ug // ncb, ug % ncb

        def idx_copy(u, b):
            j, c = unit_jc(u)
            return pltpu.make_async_copy(x_hbm.at[j, c], idx_v.at[b], isem.at[b])

        def gather(b):
            return pltpu.make_async_copy(
                lut_hbm.at[idx_v.at[b]], rows_v.at[b], gsem.at[b])

        def out_copy(u, b, r):
            j, c = unit_jc(u)
            return pltpu.make_async_copy(
                tr_v.at[b, pl.ds(r * 8, 8)], out_hbm.at[j, r, c], osem.at[b])

        for k in range(_NBUF - 1):
            idx_copy(k, k).start()
        idx_copy(0, 0).wait()
        gather(0).start()

        @pl.loop(0, upw, step=_NBUF)
        def outer(u0):
            for b in range(_NBUF):
                u = u0 + b

                @pl.when(u + _NBUF - 1 < upw)
                def _():
                    idx_copy(u + _NBUF - 1, (b + _NBUF - 1) % _NBUF).start()

                @pl.when(u + 1 < upw)
                def _():
                    nb = (b + 1) % _NBUF
                    idx_copy(u + 1, nb).wait()
                    gather(nb).start()

                gather(b).wait()

                @pl.when(u >= _NBUF)
                def _():
                    for r in range(DIM // 8):
                        out_copy(u - _NBUF, b, r).wait()

                @plsc.parallel_loop(0, DIM, unroll=8)
                def dloop(d):
                    col = jnp.full((16,), d, jnp.int32)
                    for g in range(_C // 16):
                        v = plsc.load_gather(
                            rows_v.at[b], [iota + (g * 16), col])
                        tr_v[b, d, pl.ds(g * 16, 16)] = v * SCALE

                for r in range(DIM // 8):
                    out_copy(u, b, r).start()

        for b in range(_NBUF):
            for r in range(DIM // 8):
                out_copy(upw - _NBUF + b, b, r).wait()

    return emb(xt3, lut)


def kernel(X, lut):
    s0, s1 = X.shape
    xt3 = jnp.transpose(X).astype(jnp.int32).reshape(s1, s0 // _C, _C)
    out5 = _emb_call(xt3, lut)
    out = out5.transpose(2, 4, 0, 1, 3).reshape(s0, s1, DIM)
    return out
